# baseline (speedup 1.0000x reference)
import functools

import jax
import jax.numpy as jnp
from jax import lax
from jax.experimental import pallas as pl
from jax.experimental.pallas import tpu as pltpu
from jax.experimental.pallas import tpu_sc as plsc

N = 10000
E = 320000
EPR = 32
GP = 8
EP = 327680
NSC = 2
NT = 16
NR = EP // EPR
RB = 1000

_MESH = plsc.VectorSubcoreMesh(core_axis_name="c", subcore_axis_name="s")

_RSTRIPE = 624
_RTAIL = N - NT * _RSTRIPE


def _striped_copy(src, dst, s):
    pltpu.sync_copy(src.at[pl.ds(s * _RSTRIPE, _RSTRIPE)],
                    dst.at[pl.ds(s * _RSTRIPE, _RSTRIPE)])

    @pl.when(s == NT - 1)
    def _():
        pltpu.sync_copy(src.at[pl.ds(NT * _RSTRIPE, _RTAIL)],
                        dst.at[pl.ds(NT * _RSTRIPE, _RTAIL)])


@functools.partial(
    pl.kernel,
    out_type=jax.ShapeDtypeStruct((NSC, N), jnp.float32),
    mesh=_MESH,
    scratch_types=[
        pltpu.VMEM((3, GP, EPR), jnp.int32),
        pltpu.VMEM((3, GP, EPR), jnp.float32),
        pltpu.VMEM_SHARED((N,), jnp.float32),
        pltpu.SemaphoreType.DMA((3,)),
        pltpu.SemaphoreType.DMA((GP,)),
    ],
)
def _deg_sc(dst_hbm, ew_hbm, zero_hbm, out_hbm, dst_sl, ew_sl, acc,
            isem, ssem):
    c = lax.axis_index("c")
    s = lax.axis_index("s")
    rpt = NR // (NSC * NT)
    ng = rpt // GP
    base = (c * NT + s) * rpt

    @pl.when(s == 0)
    def _():
        pltpu.sync_copy(zero_hbm, acc)

    def slab_load(g, slot):
        r0 = base + g * GP
        pltpu.async_copy(dst_hbm.at[pl.ds(r0, GP)], dst_sl.at[slot],
                         isem.at[slot])
        pltpu.async_copy(ew_hbm.at[pl.ds(r0, GP)], ew_sl.at[slot],
                         isem.at[slot])

    def slab_wait(g, slot):
        r0 = base + g * GP
        pltpu.make_async_copy(dst_hbm.at[pl.ds(r0, GP)], dst_sl.at[slot],
                              isem.at[slot]).wait()
        pltpu.make_async_copy(ew_hbm.at[pl.ds(r0, GP)], ew_sl.at[slot],
                              isem.at[slot]).wait()

    slab_load(0, 0)
    plsc.subcore_barrier()

    def group(g, carry):
        slot = g % 3
        slab_wait(g, slot)

        def drain(k, cc):
            pltpu.make_async_copy(
                ew_sl.at[slot, k], acc.at[dst_sl.at[slot, k]],
                ssem.at[k]).wait()
            return cc

        @pl.when(g > 0)
        def _():
            lax.fori_loop(0, GP, drain, 0)

        @pl.when(g + 1 < ng)
        def _():
            slab_load(g + 1, (g + 1) % 3)

        def issue(k, cc):
            pltpu.async_copy(ew_sl.at[slot, k], acc.at[dst_sl.at[slot, k]],
                             ssem.at[k], add=True)
            return cc

        lax.fori_loop(0, GP, issue, 0)
        return carry

    lax.fori_loop(0, ng, group, 0)

    def fin(k, cc):
        pltpu.make_async_copy(ew_sl.at[0, k], acc.at[dst_sl.at[0, k]],
                              ssem.at[k]).wait()
        return cc

    lax.fori_loop(0, GP, fin, 0)
    plsc.subcore_barrier()

    @pl.when(s == 0)
    def _():
        pltpu.sync_copy(acc, out_hbm.at[c])


def _make_agg(feat_split, nj):
    GA = 8
    L = 4
    scratch = [
        pltpu.VMEM((3, GA, EPR), jnp.int32),
        pltpu.VMEM((3, GA, EPR), jnp.int32),
        pltpu.VMEM((3, GA, EPR), jnp.float32),
        pltpu.VMEM((GA, EPR, 128), jnp.float32),
        pltpu.VMEM_SHARED((N, 128), jnp.float32),
        pltpu.SemaphoreType.DMA((3,)),
        pltpu.SemaphoreType.DMA((GA,)),
        pltpu.SemaphoreType.DMA((GA,)),
    ]
    if feat_split:
        scratch.insert(3, pltpu.VMEM((GA, EPR), jnp.int32))

    @functools.partial(
        pl.kernel,
        out_type=jax.ShapeDtypeStruct((NSC, N, 128), jnp.float32),
        mesh=_MESH,
        scratch_types=scratch,
    )
    def agg(hs_hbm, src_hbm, dst_hbm, ew_hbm, zero_hbm, out_hbm,
            src_sl, dst_sl, ew_sl, *rest):
        if feat_split:
            gidx, rows, acc, isem, gsem, ssem = rest
        else:
            rows, acc, isem, gsem, ssem = rest
        c = lax.axis_index("c")
        s = lax.axis_index("s")
        ntot = NR // (NT if feat_split else NSC * NT)
        nslab = ntot // GA
        base = (s if feat_split else c * NT + s) * ntot
        coff = c * N
        _striped_copy(zero_hbm, acc, s)

        def slab_load(m):
            r0 = base + m * GA
            slot = m % 3
            pltpu.async_copy(src_hbm.at[pl.ds(r0, GA)], src_sl.at[slot],
                             isem.at[slot])
            pltpu.async_copy(dst_hbm.at[pl.ds(r0, GA)], dst_sl.at[slot],
                             isem.at[slot])
            pltpu.async_copy(ew_hbm.at[pl.ds(r0, GA)], ew_sl.at[slot],
                             isem.at[slot])

        def slab_wait(m):
            r0 = base + m * GA
            slot = m % 3
            pltpu.make_async_copy(src_hbm.at[pl.ds(r0, GA)],
                                  src_sl.at[slot], isem.at[slot]).wait()
            pltpu.make_async_copy(dst_hbm.at[pl.ds(r0, GA)],
                                  dst_sl.at[slot], isem.at[slot]).wait()
            pltpu.make_async_copy(ew_hbm.at[pl.ds(r0, GA)],
                                  ew_sl.at[slot], isem.at[slot]).wait()

        def issue_gather(i):
            m = i // GA
            slot = m % 3
            r = i - m * GA
            b = i % GA
            if feat_split:
                for t in range(EPR // 16):
                    sx = pl.ds(t * 16, 16)
                    gidx[b, sx] = src_sl[slot, r, sx] + coff
                idxref = gidx.at[b]
            else:
                idxref = src_sl.at[slot, r]
            pltpu.async_copy(hs_hbm.at[idxref], rows.at[b], gsem.at[b])

        slab_load(0)
        slab_load(1)
        plsc.subcore_barrier()
        slab_wait(0)

        def prime(i, cc):
            issue_gather(i)
            return cc

        lax.fori_loop(0, L, prime, 0)

        def step(i, carry):
            j = i + L
            mj = j // GA

            @pl.when(jnp.logical_and(j - mj * GA == 0, j < ntot))
            def _():
                slab_wait(mj)

                @pl.when(mj + 1 < nslab)
                def _():
                    slab_load(mj + 1)

            @pl.when(j < ntot)
            def _():
                b = j % GA

                @pl.when(j >= GA)
                def _():
                    pltpu.make_async_copy(
                        rows.at[b], acc.at[dst_sl.at[0, 0]],
                        ssem.at[b]).wait()
                issue_gather(j)

            m = i // GA
            slot = m % 3
            r = i - m * GA
            b = i % GA
            idxref = gidx.at[b] if feat_split else src_sl.at[slot, r]
            pltpu.make_async_copy(hs_hbm.at[idxref], rows.at[b],
                                  gsem.at[b]).wait()
            for t in range(EPR // 16):
                w16 = ew_sl[slot, r, pl.ds(t * 16, 16)]
                for l in range(16):
                    w = w16[l]
                    e = t * 16 + l
                    for jj in range(nj):
                        sx = pl.ds(jj * 16, 16)
                        rows[b, e, sx] = rows[b, e, sx] * w
            pltpu.async_copy(rows.at[b], acc.at[dst_sl.at[slot, r]],
                             ssem.at[b], add=True)
            return carry

        lax.fori_loop(0, ntot, step, 0)

        def fin(k, cc):
            pltpu.make_async_copy(rows.at[k], acc.at[dst_sl.at[0, 0]],
                                  ssem.at[k]).wait()
            return cc

        lax.fori_loop(0, GA, fin, 0)
        plsc.subcore_barrier()
        _striped_copy(acc, out_hbm.at[c], s)

    return agg


_agg_e4 = _make_agg(False, 4)
_agg_e8 = _make_agg(False, 8)
_agg_f8 = _make_agg(True, 8)


@functools.partial(
    pl.kernel,
    out_type=jax.ShapeDtypeStruct((NSC, N), jnp.float32),
    mesh=_MESH,
    scratch_types=[
        pltpu.VMEM((3, GP, EPR), jnp.int32),
        pltpu.VMEM((3, GP, EPR), jnp.int32),
        pltpu.VMEM((3, GP, EPR), jnp.float32),
        pltpu.VMEM((GP, EPR), jnp.float32),
        pltpu.VMEM_SHARED((N,), jnp.float32),
        pltpu.SemaphoreType.DMA((3,)),
        pltpu.SemaphoreType.DMA((GP,)),
        pltpu.SemaphoreType.DMA((GP,)),
    ],
)
def _agg_scalar(hs_hbm, src_hbm, dst_hbm, ew_hbm, zero_hbm, out_hbm,
                src_sl, dst_sl, ew_sl, msg, acc, isem, gsem, ssem):
    c = lax.axis_index("c")
    s = lax.axis_index("s")
    rpt = NR // (NSC * NT)
    ng = rpt // GP
    base = (c * NT + s) * rpt

    @pl.when(s == 0)
    def _():
        pltpu.sync_copy(zero_hbm, acc)

    def slab_load(g, slot):
        r0 = base + g * GP
        pltpu.async_copy(src_hbm.at[pl.ds(r0, GP)], src_sl.at[slot],
                         isem.at[slot])
        pltpu.async_copy(dst_hbm.at[pl.ds(r0, GP)], dst_sl.at[slot],
                         isem.at[slot])
        pltpu.async_copy(ew_hbm.at[pl.ds(r0, GP)], ew_sl.at[slot],
                         isem.at[slot])

    def slab_wait(g, slot):
        r0 = base + g * GP
        pltpu.make_async_copy(src_hbm.at[pl.ds(r0, GP)], src_sl.at[slot],
                              isem.at[slot]).wait()
        pltpu.make_async_copy(dst_hbm.at[pl.ds(r0, GP)], dst_sl.at[slot],
                              isem.at[slot]).wait()
        pltpu.make_async_copy(ew_hbm.at[pl.ds(r0, GP)], ew_sl.at[slot],
                              isem.at[slot]).wait()

    slab_load(0, 0)
    plsc.subcore_barrier()

    def group(g, carry):
        slot = g % 3
        slab_wait(g, slot)

        @pl.when(g + 1 < ng)
        def _():
            slab_load(g + 1, (g + 1) % 3)

        def issue(k, cc):
            @pl.when(g > 0)
            def _():
                pltpu.make_async_copy(
                    msg.at[k], acc.at[dst_sl.at[slot, k]],
                    ssem.at[k]).wait()
            pltpu.async_copy(hs_hbm.at[src_sl.at[slot, k]], msg.at[k],
                             gsem.at[k])
            return cc

        lax.fori_loop(0, GP, issue, 0)

        def proc(k, cc):
            pltpu.make_async_copy(hs_hbm.at[src_sl.at[slot, k]],
                                  msg.at[k], gsem.at[k]).wait()

            def st(t, c2):
                sl = pl.ds(t * 16, 16)
                msg[k, sl] = msg[k, sl] * ew_sl[slot, k, sl]
                return c2

            lax.fori_loop(0, EPR // 16, st, 0)
            pltpu.async_copy(msg.at[k], acc.at[dst_sl.at[slot, k]],
                             ssem.at[k], add=True)
            return cc

        lax.fori_loop(0, GP, proc, 0)
        return carry

    lax.fori_loop(0, ng, group, 0)

    def fin(k, cc):
        pltpu.make_async_copy(msg.at[k], acc.at[dst_sl.at[0, k]],
                              ssem.at[k]).wait()
        return cc

    lax.fori_loop(0, GP, fin, 0)
    plsc.subcore_barrier()

    @pl.when(s == 0)
    def _():
        pltpu.sync_copy(acc, out_hbm.at[c])


def _dinv(deg_ref):
    return lax.rsqrt(deg_ref[:, 0] + deg_ref[:, 1] + 1.0)


def _tc_first(x, w, deg_t):
    din, dout = w.shape

    def body(x_ref, w_ref, deg_ref, o_ref):
        dv = _dinv(deg_ref)
        h = jnp.dot(x_ref[...], w_ref[...], preferred_element_type=jnp.float32)
        o_ref[...] = h * dv[:, None]

    return pl.pallas_call(
        body,
        grid=(N // RB,),
        in_specs=[
            pl.BlockSpec((RB, din), lambda i: (i, 0)),
            pl.BlockSpec((din, dout), lambda i: (0, 0)),
            pl.BlockSpec((RB, 2), lambda i: (i, 0)),
        ],
        out_specs=pl.BlockSpec((RB, dout), lambda i: (i, 0)),
        out_shape=jax.ShapeDtypeStruct((N, dout), jnp.float32),
    )(x, w, deg_t)


def _tc_mid(parts, hs, deg_t, b, w):
    din, dout = w.shape

    def body(p_ref, hs_ref, deg_ref, b_ref, w_ref, o_ref):
        dv = _dinv(deg_ref)
        z = p_ref[0] + p_ref[1] + hs_ref[...]
        z = jnp.maximum(z * dv[:, None] + b_ref[...], 0.0)
        h = jnp.dot(z, w_ref[...], preferred_element_type=jnp.float32)
        o_ref[...] = h * dv[:, None]

    return pl.pallas_call(
        body,
        grid=(N // RB,),
        in_specs=[
            pl.BlockSpec((2, RB, din), lambda i: (0, i, 0)),
            pl.BlockSpec((RB, din), lambda i: (i, 0)),
            pl.BlockSpec((RB, 2), lambda i: (i, 0)),
            pl.BlockSpec((din,), lambda i: (0,)),
            pl.BlockSpec((din, dout), lambda i: (0, 0)),
        ],
        out_specs=pl.BlockSpec((RB, dout), lambda i: (i, 0)),
        out_shape=jax.ShapeDtypeStruct((N, dout), jnp.float32),
    )(parts, hs, deg_t, b, w)


def _tc_mid_to_split(parts, hs, deg_t, b, w):
    din, dout = w.shape

    def body(p_ref, hs_ref, deg_ref, b_ref, w_ref, o_ref):
        dv = _dinv(deg_ref)
        z = p_ref[0] + p_ref[1] + hs_ref[...]
        z = jnp.maximum(z * dv[:, None] + b_ref[...], 0.0)
        h = jnp.dot(z, w_ref[...], preferred_element_type=jnp.float32)
        o_ref[0] = h * dv[:, None]

    return pl.pallas_call(
        body,
        grid=(N // RB, 2),
        in_specs=[
            pl.BlockSpec((2, RB, din), lambda i, cc: (0, i, 0)),
            pl.BlockSpec((RB, din), lambda i, cc: (i, 0)),
            pl.BlockSpec((RB, 2), lambda i, cc: (i, 0)),
            pl.BlockSpec((din,), lambda i, cc: (0,)),
            pl.BlockSpec((din, 128), lambda i, cc: (0, cc)),
        ],
        out_specs=pl.BlockSpec((1, RB, 128), lambda i, cc: (cc, i, 0)),
        out_shape=jax.ShapeDtypeStruct((2, N, 128), jnp.float32),
    )(parts, hs, deg_t, b, w)


def _tc_from_split(parts, hs, deg_t, b, w):
    din, dout = w.shape

    def body(p_ref, hs_ref, deg_ref, b_ref, w_ref, o_ref):
        dv = _dinv(deg_ref)
        z = jnp.concatenate(
            [p_ref[0] + hs_ref[0], p_ref[1] + hs_ref[1]], axis=-1)
        z = jnp.maximum(z * dv[:, None] + b_ref[...], 0.0)
        h = jnp.dot(z, w_ref[...], preferred_element_type=jnp.float32)
        o_ref[...] = h * dv[:, None]

    return pl.pallas_call(
        body,
        grid=(N // RB,),
        in_specs=[
            pl.BlockSpec((2, RB, 128), lambda i: (0, i, 0)),
            pl.BlockSpec((2, RB, 128), lambda i: (0, i, 0)),
            pl.BlockSpec((RB, 2), lambda i: (i, 0)),
            pl.BlockSpec((din,), lambda i: (0,)),
            pl.BlockSpec((din, dout), lambda i: (0, 0)),
        ],
        out_specs=pl.BlockSpec((RB, dout), lambda i: (i, 0)),
        out_shape=jax.ShapeDtypeStruct((N, dout), jnp.float32),
    )(parts, hs, deg_t, b, w)


def _tc_final(parts, hs, deg_t, b):
    def body(p_ref, hs_ref, deg_ref, b_ref, o_ref):
        dv = lax.rsqrt(deg_ref[:, 0] + deg_ref[:, 1] + 1.0)
        v = (p_ref[0] + p_ref[1] + hs_ref[:, 0]) * dv + b_ref[0]
        o_ref[...] = jnp.tanh(v)[:, None]

    return pl.pallas_call(
        body,
        grid=(1,),
        in_specs=[
            pl.BlockSpec((2, N), lambda i: (0, 0)),
            pl.BlockSpec((N, 1), lambda i: (0, 0)),
            pl.BlockSpec((N, 2), lambda i: (0, 0)),
            pl.BlockSpec((1,), lambda i: (0,)),
        ],
        out_specs=pl.BlockSpec((N, 1), lambda i: (0, 0)),
        out_shape=jax.ShapeDtypeStruct((N, 1), jnp.float32),
    )(parts, hs, deg_t, b)


def kernel(x, edge_index, edge_weight, W1, b1, W2, b2, W3, b3, W4, b4, W5, b5):
    ei = edge_index.astype(jnp.int32)
    npad = EP - E
    pidx = jnp.arange(npad, dtype=jnp.int32) % N
    src2 = jnp.concatenate([ei[0], pidx]).reshape(NR, EPR)
    dst2 = jnp.concatenate([ei[1], pidx]).reshape(NR, EPR)
    ew2 = jnp.concatenate(
        [edge_weight, jnp.zeros((npad,), jnp.float32)]).reshape(NR, EPR)
    z1d = jnp.zeros((N,), jnp.float32)
    z128 = jnp.zeros((N, 128), jnp.float32)

    W1p = jnp.pad(W1, ((0, 0), (0, 64)))
    b1p = jnp.pad(b1, (0, 64))
    W2p = jnp.pad(W2, ((0, 64), (0, 0)))
    W4p = jnp.pad(W4, ((0, 0), (0, 64)))
    b4p = jnp.pad(b4, (0, 64))
    W5p = jnp.pad(W5, ((0, 64), (0, 127)))

    deg_p = _deg_sc(dst2, ew2, z1d)
    deg_t = deg_p.T

    hs1 = _tc_first(x, W1p, deg_t)
    p1 = _agg_e4(hs1, src2, dst2, ew2, z128)
    hs2 = _tc_mid(p1, hs1, deg_t, b1p, W2p)
    p2 = _agg_e8(hs2, src2, dst2, ew2, z128)
    hs3 = _tc_mid_to_split(p2, hs2, deg_t, b2, W3)
    p3 = _agg_f8(hs3.reshape(2 * N, 128), src2, dst2, ew2, z128)
    hs4 = _tc_from_split(p3, hs3, deg_t, b3, W4p)
    p4 = _agg_e4(hs4, src2, dst2, ew2, z128)
    hs5f = _tc_mid(p4, hs4, deg_t, b4p, W5p)
    hs5 = hs5f[:, :1]
    p5 = _agg_scalar(hs5f[:, 0], src2, dst2, ew2, z1d)
    return _tc_final(p5, hs5, deg_t, b5)


# speedup vs baseline: 1.0075x; 1.0075x over previous
import functools

import jax
import jax.numpy as jnp
from jax import lax
from jax.experimental import pallas as pl
from jax.experimental.pallas import tpu as pltpu
from jax.experimental.pallas import tpu_sc as plsc

N = 10000
E = 320000
EPR = 32
GP = 8
EP = 327680
NSC = 2
NT = 16
NR = EP // EPR
RB = 1000

_MESH = plsc.VectorSubcoreMesh(core_axis_name="c", subcore_axis_name="s")

_RSTRIPE = 624
_RTAIL = N - NT * _RSTRIPE


def _striped_copy(src, dst, s):
    pltpu.sync_copy(src.at[pl.ds(s * _RSTRIPE, _RSTRIPE)],
                    dst.at[pl.ds(s * _RSTRIPE, _RSTRIPE)])

    @pl.when(s == NT - 1)
    def _():
        pltpu.sync_copy(src.at[pl.ds(NT * _RSTRIPE, _RTAIL)],
                        dst.at[pl.ds(NT * _RSTRIPE, _RTAIL)])


@functools.partial(
    pl.kernel,
    out_type=jax.ShapeDtypeStruct((NSC, N), jnp.float32),
    mesh=_MESH,
    scratch_types=[
        pltpu.VMEM((3, GP, EPR), jnp.int32),
        pltpu.VMEM((3, GP, EPR), jnp.float32),
        pltpu.VMEM_SHARED((N,), jnp.float32),
        pltpu.SemaphoreType.DMA((3,)),
        pltpu.SemaphoreType.DMA((GP,)),
    ],
)
def _deg_sc(dst_hbm, ew_hbm, zero_hbm, out_hbm, dst_sl, ew_sl, acc,
            isem, ssem):
    c = lax.axis_index("c")
    s = lax.axis_index("s")
    rpt = NR // (NSC * NT)
    ng = rpt // GP
    base = (c * NT + s) * rpt

    @pl.when(s == 0)
    def _():
        pltpu.sync_copy(zero_hbm, acc)

    def slab_load(g, slot):
        r0 = base + g * GP
        pltpu.async_copy(dst_hbm.at[pl.ds(r0, GP)], dst_sl.at[slot],
                         isem.at[slot])
        pltpu.async_copy(ew_hbm.at[pl.ds(r0, GP)], ew_sl.at[slot],
                         isem.at[slot])

    def slab_wait(g, slot):
        r0 = base + g * GP
        pltpu.make_async_copy(dst_hbm.at[pl.ds(r0, GP)], dst_sl.at[slot],
                              isem.at[slot]).wait()
        pltpu.make_async_copy(ew_hbm.at[pl.ds(r0, GP)], ew_sl.at[slot],
                              isem.at[slot]).wait()

    slab_load(0, 0)
    plsc.subcore_barrier()

    def group(g, carry):
        slot = g % 3
        slab_wait(g, slot)

        def drain(k, cc):
            pltpu.make_async_copy(
                ew_sl.at[slot, k], acc.at[dst_sl.at[slot, k]],
                ssem.at[k]).wait()
            return cc

        @pl.when(g > 0)
        def _():
            lax.fori_loop(0, GP, drain, 0)

        @pl.when(g + 1 < ng)
        def _():
            slab_load(g + 1, (g + 1) % 3)

        def issue(k, cc):
            pltpu.async_copy(ew_sl.at[slot, k], acc.at[dst_sl.at[slot, k]],
                             ssem.at[k], add=True)
            return cc

        lax.fori_loop(0, GP, issue, 0)
        return carry

    lax.fori_loop(0, ng, group, 0)

    def fin(k, cc):
        pltpu.make_async_copy(ew_sl.at[0, k], acc.at[dst_sl.at[0, k]],
                              ssem.at[k]).wait()
        return cc

    lax.fori_loop(0, GP, fin, 0)
    plsc.subcore_barrier()

    @pl.when(s == 0)
    def _():
        pltpu.sync_copy(acc, out_hbm.at[c])


def _make_agg(feat_split, nj):
    GA = 8
    L = 4
    scratch = [
        pltpu.VMEM((3, GA, EPR), jnp.int32),
        pltpu.VMEM((3, GA, EPR), jnp.int32),
        pltpu.VMEM((3, GA, EPR), jnp.float32),
        pltpu.VMEM((GA, EPR, 128), jnp.float32),
        pltpu.VMEM_SHARED((N, 128), jnp.float32),
        pltpu.SemaphoreType.DMA((3,)),
        pltpu.SemaphoreType.DMA((GA,)),
        pltpu.SemaphoreType.DMA((GA,)),
    ]
    if feat_split:
        scratch.insert(3, pltpu.VMEM((GA, EPR), jnp.int32))

    @functools.partial(
        pl.kernel,
        out_type=jax.ShapeDtypeStruct((NSC, N, 128), jnp.float32),
        mesh=_MESH,
        scratch_types=scratch,
    )
    def agg(hs_hbm, src_hbm, dst_hbm, ew_hbm, zero_hbm, out_hbm,
            src_sl, dst_sl, ew_sl, *rest):
        if feat_split:
            gidx, rows, acc, isem, gsem, ssem = rest
        else:
            rows, acc, isem, gsem, ssem = rest
        c = lax.axis_index("c")
        s = lax.axis_index("s")
        ntot = NR // (NT if feat_split else NSC * NT)
        nslab = ntot // GA
        base = (s if feat_split else c * NT + s) * ntot
        coff = c * N
        _striped_copy(zero_hbm, acc, s)

        def slab_load(m):
            r0 = base + m * GA
            slot = m % 3
            pltpu.async_copy(src_hbm.at[pl.ds(r0, GA)], src_sl.at[slot],
                             isem.at[slot])
            pltpu.async_copy(dst_hbm.at[pl.ds(r0, GA)], dst_sl.at[slot],
                             isem.at[slot])
            pltpu.async_copy(ew_hbm.at[pl.ds(r0, GA)], ew_sl.at[slot],
                             isem.at[slot])

        def slab_wait(m):
            r0 = base + m * GA
            slot = m % 3
            pltpu.make_async_copy(src_hbm.at[pl.ds(r0, GA)],
                                  src_sl.at[slot], isem.at[slot]).wait()
            pltpu.make_async_copy(dst_hbm.at[pl.ds(r0, GA)],
                                  dst_sl.at[slot], isem.at[slot]).wait()
            pltpu.make_async_copy(ew_hbm.at[pl.ds(r0, GA)],
                                  ew_sl.at[slot], isem.at[slot]).wait()

        def issue_gather(i):
            m = i // GA
            slot = m % 3
            r = i - m * GA
            b = i % GA
            if feat_split:
                for t in range(EPR // 16):
                    sx = pl.ds(t * 16, 16)
                    gidx[b, sx] = src_sl[slot, r, sx] + coff
                idxref = gidx.at[b]
            else:
                idxref = src_sl.at[slot, r]
            pltpu.async_copy(hs_hbm.at[idxref], rows.at[b], gsem.at[b])

        slab_load(0)
        slab_load(1)
        plsc.subcore_barrier()
        slab_wait(0)

        def prime(i, cc):
            issue_gather(i)
            return cc

        lax.fori_loop(0, L, prime, 0)

        def step(i, carry):
            j = i + L
            mj = j // GA

            @pl.when(jnp.logical_and(j - mj * GA == 0, j < ntot))
            def _():
                slab_wait(mj)

                @pl.when(mj + 1 < nslab)
                def _():
                    slab_load(mj + 1)

            @pl.when(j < ntot)
            def _():
                b = j % GA

                @pl.when(j >= GA)
                def _():
                    pltpu.make_async_copy(
                        rows.at[b], acc.at[dst_sl.at[0, 0]],
                        ssem.at[b]).wait()
                issue_gather(j)

            m = i // GA
            slot = m % 3
            r = i - m * GA
            b = i % GA
            idxref = gidx.at[b] if feat_split else src_sl.at[slot, r]
            pltpu.make_async_copy(hs_hbm.at[idxref], rows.at[b],
                                  gsem.at[b]).wait()
            for t in range(EPR // 16):
                w16 = ew_sl[slot, r, pl.ds(t * 16, 16)]
                for l in range(16):
                    w = w16[l]
                    e = t * 16 + l
                    for jj in range(nj):
                        sx = pl.ds(jj * 16, 16)
                        rows[b, e, sx] = rows[b, e, sx] * w
            pltpu.async_copy(rows.at[b], acc.at[dst_sl.at[slot, r]],
                             ssem.at[b], add=True)
            return carry

        lax.fori_loop(0, ntot, step, 0)

        def fin(k, cc):
            pltpu.make_async_copy(rows.at[k], acc.at[dst_sl.at[0, 0]],
                                  ssem.at[k]).wait()
            return cc

        lax.fori_loop(0, GA, fin, 0)
        plsc.subcore_barrier()
        _striped_copy(acc, out_hbm.at[c], s)

    return agg


_agg_e4 = _make_agg(False, 4)
_agg_e8 = _make_agg(False, 8)
_agg_f8 = _make_agg(True, 8)


@functools.partial(
    pl.kernel,
    out_type=jax.ShapeDtypeStruct((NSC, N), jnp.float32),
    mesh=_MESH,
    scratch_types=[
        pltpu.VMEM((3, GP, EPR), jnp.int32),
        pltpu.VMEM((3, GP, EPR), jnp.int32),
        pltpu.VMEM((3, GP, EPR), jnp.float32),
        pltpu.VMEM((GP, EPR), jnp.float32),
        pltpu.VMEM_SHARED((N,), jnp.float32),
        pltpu.SemaphoreType.DMA((3,)),
        pltpu.SemaphoreType.DMA((GP,)),
        pltpu.SemaphoreType.DMA((GP,)),
    ],
)
def _agg_scalar(hs_hbm, src_hbm, dst_hbm, ew_hbm, zero_hbm, out_hbm,
                src_sl, dst_sl, ew_sl, msg, acc, isem, gsem, ssem):
    c = lax.axis_index("c")
    s = lax.axis_index("s")
    rpt = NR // (NSC * NT)
    ng = rpt // GP
    base = (c * NT + s) * rpt

    @pl.when(s == 0)
    def _():
        pltpu.sync_copy(zero_hbm, acc)

    def slab_load(g, slot):
        r0 = base + g * GP
        pltpu.async_copy(src_hbm.at[pl.ds(r0, GP)], src_sl.at[slot],
                         isem.at[slot])
        pltpu.async_copy(dst_hbm.at[pl.ds(r0, GP)], dst_sl.at[slot],
                         isem.at[slot])
        pltpu.async_copy(ew_hbm.at[pl.ds(r0, GP)], ew_sl.at[slot],
                         isem.at[slot])

    def slab_wait(g, slot):
        r0 = base + g * GP
        pltpu.make_async_copy(src_hbm.at[pl.ds(r0, GP)], src_sl.at[slot],
                              isem.at[slot]).wait()
        pltpu.make_async_copy(dst_hbm.at[pl.ds(r0, GP)], dst_sl.at[slot],
                              isem.at[slot]).wait()
        pltpu.make_async_copy(ew_hbm.at[pl.ds(r0, GP)], ew_sl.at[slot],
                              isem.at[slot]).wait()

    ntot = rpt
    nslab = ntot // GP
    L5 = 4

    def issue_gather(i):
        m = i // GP
        slot = m % 3
        r = i - m * GP
        b = i % GP
        pltpu.async_copy(hs_hbm.at[src_sl.at[slot, r]], msg.at[b],
                         gsem.at[b])

    slab_load(0, 0)
    slab_load(1, 1)
    plsc.subcore_barrier()
    slab_wait(0, 0)

    def prime(i, cc):
        issue_gather(i)
        return cc

    lax.fori_loop(0, L5, prime, 0)

    def step(i, carry):
        j = i + L5
        mj = j // GP

        @pl.when(jnp.logical_and(j - mj * GP == 0, j < ntot))
        def _():
            slab_wait(mj, mj % 3)

            @pl.when(mj + 1 < nslab)
            def _():
                slab_load(mj + 1, (mj + 1) % 3)

        @pl.when(j < ntot)
        def _():
            b = j % GP

            @pl.when(j >= GP)
            def _():
                pltpu.make_async_copy(msg.at[b], acc.at[dst_sl.at[0, 0]],
                                      ssem.at[b]).wait()
            issue_gather(j)

        m = i // GP
        slot = m % 3
        r = i - m * GP
        b = i % GP
        pltpu.make_async_copy(hs_hbm.at[src_sl.at[slot, r]], msg.at[b],
                              gsem.at[b]).wait()
        for t in range(EPR // 16):
            sl = pl.ds(t * 16, 16)
            msg[b, sl] = msg[b, sl] * ew_sl[slot, r, sl]
        pltpu.async_copy(msg.at[b], acc.at[dst_sl.at[slot, r]],
                         ssem.at[b], add=True)
        return carry

    lax.fori_loop(0, ntot, step, 0)

    def fin(k, cc):
        pltpu.make_async_copy(msg.at[k], acc.at[dst_sl.at[0, 0]],
                              ssem.at[k]).wait()
        return cc

    lax.fori_loop(0, GP, fin, 0)
    plsc.subcore_barrier()

    @pl.when(s == 0)
    def _():
        pltpu.sync_copy(acc, out_hbm.at[c])


def _dinv(deg_ref):
    return lax.rsqrt(deg_ref[:, 0] + deg_ref[:, 1] + 1.0)


def _tc_first(x, w, deg_t):
    din, dout = w.shape

    def body(x_ref, w_ref, deg_ref, o_ref):
        dv = _dinv(deg_ref)
        h = jnp.dot(x_ref[...], w_ref[...], preferred_element_type=jnp.float32)
        o_ref[...] = h * dv[:, None]

    return pl.pallas_call(
        body,
        grid=(N // RB,),
        in_specs=[
            pl.BlockSpec((RB, din), lambda i: (i, 0)),
            pl.BlockSpec((din, dout), lambda i: (0, 0)),
            pl.BlockSpec((RB, 2), lambda i: (i, 0)),
        ],
        out_specs=pl.BlockSpec((RB, dout), lambda i: (i, 0)),
        out_shape=jax.ShapeDtypeStruct((N, dout), jnp.float32),
    )(x, w, deg_t)


def _tc_mid(parts, hs, deg_t, b, w):
    din, dout = w.shape

    def body(p_ref, hs_ref, deg_ref, b_ref, w_ref, o_ref):
        dv = _dinv(deg_ref)
        z = p_ref[0] + p_ref[1] + hs_ref[...]
        z = jnp.maximum(z * dv[:, None] + b_ref[...], 0.0)
        h = jnp.dot(z, w_ref[...], preferred_element_type=jnp.float32)
        o_ref[...] = h * dv[:, None]

    return pl.pallas_call(
        body,
        grid=(N // RB,),
        in_specs=[
            pl.BlockSpec((2, RB, din), lambda i: (0, i, 0)),
            pl.BlockSpec((RB, din), lambda i: (i, 0)),
            pl.BlockSpec((RB, 2), lambda i: (i, 0)),
            pl.BlockSpec((din,), lambda i: (0,)),
            pl.BlockSpec((din, dout), lambda i: (0, 0)),
        ],
        out_specs=pl.BlockSpec((RB, dout), lambda i: (i, 0)),
        out_shape=jax.ShapeDtypeStruct((N, dout), jnp.float32),
    )(parts, hs, deg_t, b, w)


def _tc_mid_to_split(parts, hs, deg_t, b, w):
    din, dout = w.shape

    def body(p_ref, hs_ref, deg_ref, b_ref, w_ref, o_ref):
        dv = _dinv(deg_ref)
        z = p_ref[0] + p_ref[1] + hs_ref[...]
        z = jnp.maximum(z * dv[:, None] + b_ref[...], 0.0)
        h = jnp.dot(z, w_ref[...], preferred_element_type=jnp.float32)
        o_ref[0] = h * dv[:, None]

    return pl.pallas_call(
        body,
        grid=(N // RB, 2),
        in_specs=[
            pl.BlockSpec((2, RB, din), lambda i, cc: (0, i, 0)),
            pl.BlockSpec((RB, din), lambda i, cc: (i, 0)),
            pl.BlockSpec((RB, 2), lambda i, cc: (i, 0)),
            pl.BlockSpec((din,), lambda i, cc: (0,)),
            pl.BlockSpec((din, 128), lambda i, cc: (0, cc)),
        ],
        out_specs=pl.BlockSpec((1, RB, 128), lambda i, cc: (cc, i, 0)),
        out_shape=jax.ShapeDtypeStruct((2, N, 128), jnp.float32),
    )(parts, hs, deg_t, b, w)


def _tc_from_split(parts, hs, deg_t, b, w):
    din, dout = w.shape

    def body(p_ref, hs_ref, deg_ref, b_ref, w_ref, o_ref):
        dv = _dinv(deg_ref)
        z = jnp.concatenate(
            [p_ref[0] + hs_ref[0], p_ref[1] + hs_ref[1]], axis=-1)
        z = jnp.maximum(z * dv[:, None] + b_ref[...], 0.0)
        h = jnp.dot(z, w_ref[...], preferred_element_type=jnp.float32)
        o_ref[...] = h * dv[:, None]

    return pl.pallas_call(
        body,
        grid=(N // RB,),
        in_specs=[
            pl.BlockSpec((2, RB, 128), lambda i: (0, i, 0)),
            pl.BlockSpec((2, RB, 128), lambda i: (0, i, 0)),
            pl.BlockSpec((RB, 2), lambda i: (i, 0)),
            pl.BlockSpec((din,), lambda i: (0,)),
            pl.BlockSpec((din, dout), lambda i: (0, 0)),
        ],
        out_specs=pl.BlockSpec((RB, dout), lambda i: (i, 0)),
        out_shape=jax.ShapeDtypeStruct((N, dout), jnp.float32),
    )(parts, hs, deg_t, b, w)


def _tc_final(parts, hs, deg_t, b):
    def body(p_ref, hs_ref, deg_ref, b_ref, o_ref):
        dv = lax.rsqrt(deg_ref[:, 0] + deg_ref[:, 1] + 1.0)
        v = (p_ref[0] + p_ref[1] + hs_ref[:, 0]) * dv + b_ref[0]
        o_ref[...] = jnp.tanh(v)[:, None]

    return pl.pallas_call(
        body,
        grid=(1,),
        in_specs=[
            pl.BlockSpec((2, N), lambda i: (0, 0)),
            pl.BlockSpec((N, 1), lambda i: (0, 0)),
            pl.BlockSpec((N, 2), lambda i: (0, 0)),
            pl.BlockSpec((1,), lambda i: (0,)),
        ],
        out_specs=pl.BlockSpec((N, 1), lambda i: (0, 0)),
        out_shape=jax.ShapeDtypeStruct((N, 1), jnp.float32),
    )(parts, hs, deg_t, b)


def kernel(x, edge_index, edge_weight, W1, b1, W2, b2, W3, b3, W4, b4, W5, b5):
    ei = edge_index.astype(jnp.int32)
    npad = EP - E
    pidx = jnp.arange(npad, dtype=jnp.int32) % N
    src2 = jnp.concatenate([ei[0], pidx]).reshape(NR, EPR)
    dst2 = jnp.concatenate([ei[1], pidx]).reshape(NR, EPR)
    ew2 = jnp.concatenate(
        [edge_weight, jnp.zeros((npad,), jnp.float32)]).reshape(NR, EPR)
    z1d = jnp.zeros((N,), jnp.float32)
    z128 = jnp.zeros((N, 128), jnp.float32)

    W1p = jnp.pad(W1, ((0, 0), (0, 64)))
    b1p = jnp.pad(b1, (0, 64))
    W2p = jnp.pad(W2, ((0, 64), (0, 0)))
    W4p = jnp.pad(W4, ((0, 0), (0, 64)))
    b4p = jnp.pad(b4, (0, 64))
    W5p = jnp.pad(W5, ((0, 64), (0, 127)))

    deg_p = _deg_sc(dst2, ew2, z1d)
    deg_t = deg_p.T

    hs1 = _tc_first(x, W1p, deg_t)
    p1 = _agg_e4(hs1, src2, dst2, ew2, z128)
    hs2 = _tc_mid(p1, hs1, deg_t, b1p, W2p)
    p2 = _agg_e8(hs2, src2, dst2, ew2, z128)
    hs3 = _tc_mid_to_split(p2, hs2, deg_t, b2, W3)
    p3 = _agg_f8(hs3.reshape(2 * N, 128), src2, dst2, ew2, z128)
    hs4 = _tc_from_split(p3, hs3, deg_t, b3, W4p)
    p4 = _agg_e4(hs4, src2, dst2, ew2, z128)
    hs5f = _tc_mid(p4, hs4, deg_t, b4p, W5p)
    hs5 = hs5f[:, :1]
    p5 = _agg_scalar(hs5f[:, 0], src2, dst2, ew2, z1d)
    return _tc_final(p5, hs5, deg_t, b5)


# speedup vs baseline: 1.0645x; 1.0566x over previous
import functools

import jax
import jax.numpy as jnp
from jax import lax
from jax.experimental import pallas as pl
from jax.experimental.pallas import tpu as pltpu
from jax.experimental.pallas import tpu_sc as plsc

N = 10000
E = 320000
EPR = 64
GP = 8
EP = 327680
NSC = 2
NT = 16
NR = EP // EPR
RB = 1000

_MESH = plsc.VectorSubcoreMesh(core_axis_name="c", subcore_axis_name="s")

_RSTRIPE = 624
_RTAIL = N - NT * _RSTRIPE


def _striped_copy(src, dst, s):
    pltpu.sync_copy(src.at[pl.ds(s * _RSTRIPE, _RSTRIPE)],
                    dst.at[pl.ds(s * _RSTRIPE, _RSTRIPE)])

    @pl.when(s == NT - 1)
    def _():
        pltpu.sync_copy(src.at[pl.ds(NT * _RSTRIPE, _RTAIL)],
                        dst.at[pl.ds(NT * _RSTRIPE, _RTAIL)])


@functools.partial(
    pl.kernel,
    out_type=jax.ShapeDtypeStruct((NSC, N), jnp.float32),
    mesh=_MESH,
    scratch_types=[
        pltpu.VMEM((3, GP, EPR), jnp.int32),
        pltpu.VMEM((3, GP, EPR), jnp.float32),
        pltpu.VMEM_SHARED((N,), jnp.float32),
        pltpu.SemaphoreType.DMA((3,)),
        pltpu.SemaphoreType.DMA((GP,)),
    ],
)
def _deg_sc(dst_hbm, ew_hbm, zero_hbm, out_hbm, dst_sl, ew_sl, acc,
            isem, ssem):
    c = lax.axis_index("c")
    s = lax.axis_index("s")
    rpt = NR // (NSC * NT)
    ng = rpt // GP
    base = (c * NT + s) * rpt

    @pl.when(s == 0)
    def _():
        pltpu.sync_copy(zero_hbm, acc)

    def slab_load(g, slot):
        r0 = base + g * GP
        pltpu.async_copy(dst_hbm.at[pl.ds(r0, GP)], dst_sl.at[slot],
                         isem.at[slot])
        pltpu.async_copy(ew_hbm.at[pl.ds(r0, GP)], ew_sl.at[slot],
                         isem.at[slot])

    def slab_wait(g, slot):
        r0 = base + g * GP
        pltpu.make_async_copy(dst_hbm.at[pl.ds(r0, GP)], dst_sl.at[slot],
                              isem.at[slot]).wait()
        pltpu.make_async_copy(ew_hbm.at[pl.ds(r0, GP)], ew_sl.at[slot],
                              isem.at[slot]).wait()

    slab_load(0, 0)
    plsc.subcore_barrier()

    def group(g, carry):
        slot = g % 3
        slab_wait(g, slot)

        def drain(k, cc):
            pltpu.make_async_copy(
                ew_sl.at[slot, k], acc.at[dst_sl.at[slot, k]],
                ssem.at[k]).wait()
            return cc

        @pl.when(g > 0)
        def _():
            lax.fori_loop(0, GP, drain, 0)

        @pl.when(g + 1 < ng)
        def _():
            slab_load(g + 1, (g + 1) % 3)

        def issue(k, cc):
            pltpu.async_copy(ew_sl.at[slot, k], acc.at[dst_sl.at[slot, k]],
                             ssem.at[k], add=True)
            return cc

        lax.fori_loop(0, GP, issue, 0)
        return carry

    lax.fori_loop(0, ng, group, 0)

    def fin(k, cc):
        pltpu.make_async_copy(ew_sl.at[0, k], acc.at[dst_sl.at[0, k]],
                              ssem.at[k]).wait()
        return cc

    lax.fori_loop(0, GP, fin, 0)
    plsc.subcore_barrier()

    @pl.when(s == 0)
    def _():
        pltpu.sync_copy(acc, out_hbm.at[c])


def _make_agg(feat_split, nj):
    SB = 8
    GA = 5
    L = 3
    scratch = [
        pltpu.VMEM((3, SB, EPR), jnp.int32),
        pltpu.VMEM((3, SB, EPR), jnp.int32),
        pltpu.VMEM((3, SB, EPR), jnp.float32),
        pltpu.VMEM((GA, EPR, 128), jnp.float32),
        pltpu.VMEM_SHARED((N, 128), jnp.float32),
        pltpu.SemaphoreType.DMA((3,)),
        pltpu.SemaphoreType.DMA((GA,)),
        pltpu.SemaphoreType.DMA((GA,)),
    ]
    @functools.partial(
        pl.kernel,
        out_type=jax.ShapeDtypeStruct((NSC, N, 128), jnp.float32),
        mesh=_MESH,
        scratch_types=scratch,
    )
    def agg(hs_hbm, src_hbm, dst_hbm, ew_hbm, zero_hbm, out_hbm,
            src_sl, dst_sl, ew_sl, rows, acc, isem, gsem, ssem):
        c = lax.axis_index("c")
        s = lax.axis_index("s")
        ntot = NR // (NT if feat_split else NSC * NT)
        nslab = ntot // SB
        base = (s if feat_split else c * NT + s) * ntot

        def src_view(r0):
            if feat_split:
                return src_hbm.at[c, pl.ds(r0, SB)]
            return src_hbm.at[pl.ds(r0, SB)]

        _striped_copy(zero_hbm, acc, s)

        def slab_load(m):
            r0 = base + m * SB
            slot = m % 3
            pltpu.async_copy(src_view(r0), src_sl.at[slot], isem.at[slot])
            pltpu.async_copy(dst_hbm.at[pl.ds(r0, SB)], dst_sl.at[slot],
                             isem.at[slot])
            pltpu.async_copy(ew_hbm.at[pl.ds(r0, SB)], ew_sl.at[slot],
                             isem.at[slot])

        def slab_wait(m):
            r0 = base + m * SB
            slot = m % 3
            pltpu.make_async_copy(src_view(r0), src_sl.at[slot],
                                  isem.at[slot]).wait()
            pltpu.make_async_copy(dst_hbm.at[pl.ds(r0, SB)],
                                  dst_sl.at[slot], isem.at[slot]).wait()
            pltpu.make_async_copy(ew_hbm.at[pl.ds(r0, SB)],
                                  ew_sl.at[slot], isem.at[slot]).wait()

        def issue_gather(i):
            m = i // SB
            slot = m % 3
            r = i - m * SB
            b = i % GA
            pltpu.async_copy(hs_hbm.at[src_sl.at[slot, r]], rows.at[b],
                             gsem.at[b])

        slab_load(0)
        slab_load(1)
        plsc.subcore_barrier()
        slab_wait(0)

        def prime(i, cc):
            issue_gather(i)
            return cc

        lax.fori_loop(0, L, prime, 0)

        def step(i, carry):
            j = i + L
            mj = j // SB

            @pl.when(jnp.logical_and(j - mj * SB == 0, j < ntot))
            def _():
                slab_wait(mj)

                @pl.when(mj + 1 < nslab)
                def _():
                    slab_load(mj + 1)

            @pl.when(j < ntot)
            def _():
                b = j % GA

                @pl.when(j >= GA)
                def _():
                    pltpu.make_async_copy(
                        rows.at[b], acc.at[dst_sl.at[0, 0]],
                        ssem.at[b]).wait()
                issue_gather(j)

            m = i // SB
            slot = m % 3
            r = i - m * SB
            b = i % GA
            pltpu.make_async_copy(hs_hbm.at[src_sl.at[slot, r]],
                                  rows.at[b], gsem.at[b]).wait()
            for t in range(EPR // 16):
                w16 = ew_sl[slot, r, pl.ds(t * 16, 16)]
                for l in range(16):
                    w = w16[l]
                    e = t * 16 + l
                    for jj in range(nj):
                        sx = pl.ds(jj * 16, 16)
                        rows[b, e, sx] = rows[b, e, sx] * w
            pltpu.async_copy(rows.at[b], acc.at[dst_sl.at[slot, r]],
                             ssem.at[b], add=True)
            return carry

        lax.fori_loop(0, ntot, step, 0)

        def fin(k, cc):
            pltpu.make_async_copy(rows.at[k], acc.at[dst_sl.at[0, 0]],
                                  ssem.at[k]).wait()
            return cc

        lax.fori_loop(0, GA, fin, 0)
        plsc.subcore_barrier()
        _striped_copy(acc, out_hbm.at[c], s)

    return agg


_agg_e4 = _make_agg(False, 4)
_agg_e8 = _make_agg(False, 8)
_agg_f8 = _make_agg(True, 8)


@functools.partial(
    pl.kernel,
    out_type=jax.ShapeDtypeStruct((NSC, N), jnp.float32),
    mesh=_MESH,
    scratch_types=[
        pltpu.VMEM((3, GP, EPR), jnp.int32),
        pltpu.VMEM((3, GP, EPR), jnp.int32),
        pltpu.VMEM((3, GP, EPR), jnp.float32),
        pltpu.VMEM((GP, EPR), jnp.float32),
        pltpu.VMEM_SHARED((N,), jnp.float32),
        pltpu.SemaphoreType.DMA((3,)),
        pltpu.SemaphoreType.DMA((GP,)),
        pltpu.SemaphoreType.DMA((GP,)),
    ],
)
def _agg_scalar(hs_hbm, src_hbm, dst_hbm, ew_hbm, zero_hbm, out_hbm,
                src_sl, dst_sl, ew_sl, msg, acc, isem, gsem, ssem):
    c = lax.axis_index("c")
    s = lax.axis_index("s")
    rpt = NR // (NSC * NT)
    ng = rpt // GP
    base = (c * NT + s) * rpt

    @pl.when(s == 0)
    def _():
        pltpu.sync_copy(zero_hbm, acc)

    def slab_load(g, slot):
        r0 = base + g * GP
        pltpu.async_copy(src_hbm.at[pl.ds(r0, GP)], src_sl.at[slot],
                         isem.at[slot])
        pltpu.async_copy(dst_hbm.at[pl.ds(r0, GP)], dst_sl.at[slot],
                         isem.at[slot])
        pltpu.async_copy(ew_hbm.at[pl.ds(r0, GP)], ew_sl.at[slot],
                         isem.at[slot])

    def slab_wait(g, slot):
        r0 = base + g * GP
        pltpu.make_async_copy(src_hbm.at[pl.ds(r0, GP)], src_sl.at[slot],
                              isem.at[slot]).wait()
        pltpu.make_async_copy(dst_hbm.at[pl.ds(r0, GP)], dst_sl.at[slot],
                              isem.at[slot]).wait()
        pltpu.make_async_copy(ew_hbm.at[pl.ds(r0, GP)], ew_sl.at[slot],
                              isem.at[slot]).wait()

    ntot = rpt
    nslab = ntot // GP
    L5 = 4

    def issue_gather(i):
        m = i // GP
        slot = m % 3
        r = i - m * GP
        b = i % GP
        pltpu.async_copy(hs_hbm.at[src_sl.at[slot, r]], msg.at[b],
                         gsem.at[b])

    slab_load(0, 0)
    slab_load(1, 1)
    plsc.subcore_barrier()
    slab_wait(0, 0)

    def prime(i, cc):
        issue_gather(i)
        return cc

    lax.fori_loop(0, L5, prime, 0)

    def step(i, carry):
        j = i + L5
        mj = j // GP

        @pl.when(jnp.logical_and(j - mj * GP == 0, j < ntot))
        def _():
            slab_wait(mj, mj % 3)

            @pl.when(mj + 1 < nslab)
            def _():
                slab_load(mj + 1, (mj + 1) % 3)

        @pl.when(j < ntot)
        def _():
            b = j % GP

            @pl.when(j >= GP)
            def _():
                pltpu.make_async_copy(msg.at[b], acc.at[dst_sl.at[0, 0]],
                                      ssem.at[b]).wait()
            issue_gather(j)

        m = i // GP
        slot = m % 3
        r = i - m * GP
        b = i % GP
        pltpu.make_async_copy(hs_hbm.at[src_sl.at[slot, r]], msg.at[b],
                              gsem.at[b]).wait()
        for t in range(EPR // 16):
            sl = pl.ds(t * 16, 16)
            msg[b, sl] = msg[b, sl] * ew_sl[slot, r, sl]
        pltpu.async_copy(msg.at[b], acc.at[dst_sl.at[slot, r]],
                         ssem.at[b], add=True)
        return carry

    lax.fori_loop(0, ntot, step, 0)

    def fin(k, cc):
        pltpu.make_async_copy(msg.at[k], acc.at[dst_sl.at[0, 0]],
                              ssem.at[k]).wait()
        return cc

    lax.fori_loop(0, GP, fin, 0)
    plsc.subcore_barrier()

    @pl.when(s == 0)
    def _():
        pltpu.sync_copy(acc, out_hbm.at[c])


def _dinv(deg_ref):
    return lax.rsqrt(deg_ref[:, 0] + deg_ref[:, 1] + 1.0)


def _tc_first(x, w, deg_t):
    din, dout = w.shape

    def body(x_ref, w_ref, deg_ref, o_ref):
        dv = _dinv(deg_ref)
        h = jnp.dot(x_ref[...], w_ref[...], preferred_element_type=jnp.float32)
        o_ref[...] = h * dv[:, None]

    return pl.pallas_call(
        body,
        grid=(N // RB,),
        in_specs=[
            pl.BlockSpec((RB, din), lambda i: (i, 0)),
            pl.BlockSpec((din, dout), lambda i: (0, 0)),
            pl.BlockSpec((RB, 2), lambda i: (i, 0)),
        ],
        out_specs=pl.BlockSpec((RB, dout), lambda i: (i, 0)),
        out_shape=jax.ShapeDtypeStruct((N, dout), jnp.float32),
    )(x, w, deg_t)


def _tc_mid(parts, hs, deg_t, b, w):
    din, dout = w.shape

    def body(p_ref, hs_ref, deg_ref, b_ref, w_ref, o_ref):
        dv = _dinv(deg_ref)
        z = p_ref[0] + p_ref[1] + hs_ref[...]
        z = jnp.maximum(z * dv[:, None] + b_ref[...], 0.0)
        h = jnp.dot(z, w_ref[...], preferred_element_type=jnp.float32)
        o_ref[...] = h * dv[:, None]

    return pl.pallas_call(
        body,
        grid=(N // RB,),
        in_specs=[
            pl.BlockSpec((2, RB, din), lambda i: (0, i, 0)),
            pl.BlockSpec((RB, din), lambda i: (i, 0)),
            pl.BlockSpec((RB, 2), lambda i: (i, 0)),
            pl.BlockSpec((din,), lambda i: (0,)),
            pl.BlockSpec((din, dout), lambda i: (0, 0)),
        ],
        out_specs=pl.BlockSpec((RB, dout), lambda i: (i, 0)),
        out_shape=jax.ShapeDtypeStruct((N, dout), jnp.float32),
    )(parts, hs, deg_t, b, w)


def _tc_mid_to_split(parts, hs, deg_t, b, w):
    din, dout = w.shape

    def body(p_ref, hs_ref, deg_ref, b_ref, w_ref, o_ref):
        dv = _dinv(deg_ref)
        z = p_ref[0] + p_ref[1] + hs_ref[...]
        z = jnp.maximum(z * dv[:, None] + b_ref[...], 0.0)
        h = jnp.dot(z, w_ref[...], preferred_element_type=jnp.float32)
        o_ref[0] = h * dv[:, None]

    return pl.pallas_call(
        body,
        grid=(N // RB, 2),
        in_specs=[
            pl.BlockSpec((2, RB, din), lambda i, cc: (0, i, 0)),
            pl.BlockSpec((RB, din), lambda i, cc: (i, 0)),
            pl.BlockSpec((RB, 2), lambda i, cc: (i, 0)),
            pl.BlockSpec((din,), lambda i, cc: (0,)),
            pl.BlockSpec((din, 128), lambda i, cc: (0, cc)),
        ],
        out_specs=pl.BlockSpec((1, RB, 128), lambda i, cc: (cc, i, 0)),
        out_shape=jax.ShapeDtypeStruct((2, N, 128), jnp.float32),
    )(parts, hs, deg_t, b, w)


def _tc_from_split(parts, hs, deg_t, b, w):
    din, dout = w.shape

    def body(p_ref, hs_ref, deg_ref, b_ref, w_ref, o_ref):
        dv = _dinv(deg_ref)
        z = jnp.concatenate(
            [p_ref[0] + hs_ref[0], p_ref[1] + hs_ref[1]], axis=-1)
        z = jnp.maximum(z * dv[:, None] + b_ref[...], 0.0)
        h = jnp.dot(z, w_ref[...], preferred_element_type=jnp.float32)
        o_ref[...] = h * dv[:, None]

    return pl.pallas_call(
        body,
        grid=(N // RB,),
        in_specs=[
            pl.BlockSpec((2, RB, 128), lambda i: (0, i, 0)),
            pl.BlockSpec((2, RB, 128), lambda i: (0, i, 0)),
            pl.BlockSpec((RB, 2), lambda i: (i, 0)),
            pl.BlockSpec((din,), lambda i: (0,)),
            pl.BlockSpec((din, dout), lambda i: (0, 0)),
        ],
        out_specs=pl.BlockSpec((RB, dout), lambda i: (i, 0)),
        out_shape=jax.ShapeDtypeStruct((N, dout), jnp.float32),
    )(parts, hs, deg_t, b, w)


def _tc_final(parts, hs, deg_t, b):
    def body(p_ref, hs_ref, deg_ref, b_ref, o_ref):
        dv = lax.rsqrt(deg_ref[:, 0] + deg_ref[:, 1] + 1.0)
        v = (p_ref[0] + p_ref[1] + hs_ref[:, 0]) * dv + b_ref[0]
        o_ref[...] = jnp.tanh(v)[:, None]

    return pl.pallas_call(
        body,
        grid=(1,),
        in_specs=[
            pl.BlockSpec((2, N), lambda i: (0, 0)),
            pl.BlockSpec((N, 1), lambda i: (0, 0)),
            pl.BlockSpec((N, 2), lambda i: (0, 0)),
            pl.BlockSpec((1,), lambda i: (0,)),
        ],
        out_specs=pl.BlockSpec((N, 1), lambda i: (0, 0)),
        out_shape=jax.ShapeDtypeStruct((N, 1), jnp.float32),
    )(parts, hs, deg_t, b)


def kernel(x, edge_index, edge_weight, W1, b1, W2, b2, W3, b3, W4, b4, W5, b5):
    ei = edge_index.astype(jnp.int32)
    npad = EP - E
    pidx = jnp.arange(npad, dtype=jnp.int32) % N
    src2 = jnp.concatenate([ei[0], pidx]).reshape(NR, EPR)
    dst2 = jnp.concatenate([ei[1], pidx]).reshape(NR, EPR)
    ew2 = jnp.concatenate(
        [edge_weight, jnp.zeros((npad,), jnp.float32)]).reshape(NR, EPR)
    z1d = jnp.zeros((N,), jnp.float32)
    z128 = jnp.zeros((N, 128), jnp.float32)

    W1p = jnp.pad(W1, ((0, 0), (0, 64)))
    b1p = jnp.pad(b1, (0, 64))
    W2p = jnp.pad(W2, ((0, 64), (0, 0)))
    W4p = jnp.pad(W4, ((0, 0), (0, 64)))
    b4p = jnp.pad(b4, (0, 64))
    W5p = jnp.pad(W5, ((0, 64), (0, 127)))

    deg_p = _deg_sc(dst2, ew2, z1d)
    deg_t = deg_p.T

    hs1 = _tc_first(x, W1p, deg_t)
    p1 = _agg_e4(hs1, src2, dst2, ew2, z128)
    hs2 = _tc_mid(p1, hs1, deg_t, b1p, W2p)
    p2 = _agg_e8(hs2, src2, dst2, ew2, z128)
    hs3 = _tc_mid_to_split(p2, hs2, deg_t, b2, W3)
    src2x = jnp.stack([src2, src2 + N])
    p3 = _agg_f8(hs3.reshape(2 * N, 128), src2x, dst2, ew2, z128)
    hs4 = _tc_from_split(p3, hs3, deg_t, b3, W4p)
    p4 = _agg_e4(hs4, src2, dst2, ew2, z128)
    hs5f = _tc_mid(p4, hs4, deg_t, b4p, W5p)
    hs5 = hs5f[:, :1]
    p5 = _agg_scalar(hs5f[:, 0], src2, dst2, ew2, z1d)
    return _tc_final(p5, hs5, deg_t, b5)
